# baseline (speedup 1.0000x reference)
import jax
import jax.numpy as jnp
from jax.experimental import pallas as pl

D = 64
BLK = 256


def _score_loss_kernel(hp_ref, tp_ref, rp_ref, hn_ref, tn_ref, rn_ref, out_ref):
    def score(h_ref, t_ref, r_ref):
        h = h_ref[0]
        t = t_ref[0]
        r = r_ref[0]
        prod = r * h[:, :, None] * t[:, None, :]
        return jnp.sum(prod, axis=(1, 2)) * (1.0 / D)

    sp = score(hp_ref, tp_ref, rp_ref)
    sn = score(hn_ref, tn_ref, rn_ref)
    part = jnp.sum(jnp.maximum(0.0, sn - sp + 1.0)).reshape(1, 1)

    @pl.when(pl.program_id(0) == 0)
    def _():
        out_ref[...] = jnp.zeros((1, 1), jnp.float32)

    out_ref[...] += part


def kernel(ph, pt, pr, nh, nt, nr, ent_embeddings, rel_matrices):
    B = ph.shape[0]
    hp = jnp.take(ent_embeddings, ph, axis=0)
    tp = jnp.take(ent_embeddings, pt, axis=0)
    hn = jnp.take(ent_embeddings, nh, axis=0)
    tn = jnp.take(ent_embeddings, nt, axis=0)
    rp = jnp.take(rel_matrices, pr, axis=0).reshape(B, D, D)
    rn = jnp.take(rel_matrices, nr, axis=0).reshape(B, D, D)

    nblk = B // BLK
    vec_spec = pl.BlockSpec((1, BLK, D), lambda i: (i, 0, 0))
    mat_spec = pl.BlockSpec((1, BLK, D, D), lambda i: (i, 0, 0, 0))
    out = pl.pallas_call(
        _score_loss_kernel,
        grid=(nblk,),
        in_specs=[vec_spec, vec_spec, mat_spec, vec_spec, vec_spec, mat_spec],
        out_specs=pl.BlockSpec((1, 1), lambda i: (0, 0)),
        out_shape=jax.ShapeDtypeStruct((1, 1), jnp.float32),
    )(
        hp.reshape(nblk, BLK, D),
        tp.reshape(nblk, BLK, D),
        rp.reshape(nblk, BLK, D, D),
        hn.reshape(nblk, BLK, D),
        tn.reshape(nblk, BLK, D),
        rn.reshape(nblk, BLK, D, D),
    )
    return out[0, 0]
